# Initial kernel scaffold; baseline (speedup 1.0000x reference)
import jax
import jax.numpy as jnp
from jax.experimental import pallas as pl


def kernel(x, adj, weight, bias):
    raise NotImplementedError("write your pallas kernel here")


# speedup vs baseline: 3.2999x; 3.2999x over previous
import jax
import jax.numpy as jnp
from jax.experimental import pallas as pl
from jax.experimental.pallas import tpu as pltpu


def _round_up(a: int, b: int) -> int:
    return ((a + b - 1) // b) * b


def _xw_kernel(x_ref, w_ref, s_ref):
    s_ref[...] = jnp.dot(
        x_ref[...].astype(jnp.bfloat16),
        w_ref[...].astype(jnp.bfloat16),
        preferred_element_type=jnp.float32,
    ).astype(s_ref.dtype)


def _prop_kernel(adj_ref, s_ref, b_ref, out_ref):
    a = adj_ref[...].astype(jnp.bfloat16)
    acc = jnp.dot(a, s_ref[...], preferred_element_type=jnp.float32)
    out_ref[...] = (acc + b_ref[...]).astype(out_ref.dtype)


def kernel(x, adj, weight, bias):
    N, f_in = x.shape
    f_in_w, f_out = weight.shape
    assert f_in == f_in_w, "weight shape mismatch"
    assert adj.shape == (N, N), "adj must be [N, N]"

    out_dtype = x.dtype

    N_pad = _round_up(N, 256)
    f_out_pad = _round_up(f_out, 128)

    x_p = jnp.pad(x, ((0, N_pad - N), (0, 0)))
    adj_p = jnp.pad(adj, ((0, N_pad - N), (0, N_pad - N)))
    w_p = jnp.pad(weight, ((0, 0), (0, f_out_pad - f_out)))
    b = bias if bias is not None else jnp.zeros((f_out,), out_dtype)
    b_p = jnp.pad(b, (0, f_out_pad - f_out)).reshape(1, f_out_pad).astype(jnp.float32)

    tm1 = 512 if N_pad % 512 == 0 else 256
    tm2 = 256

    support = pl.pallas_call(
        _xw_kernel,
        out_shape=jax.ShapeDtypeStruct((N_pad, f_out_pad), jnp.bfloat16),
        grid=(N_pad // tm1,),
        in_specs=[
            pl.BlockSpec((tm1, f_in), lambda i: (i, 0)),
            pl.BlockSpec((f_in, f_out_pad), lambda i: (0, 0)),
        ],
        out_specs=pl.BlockSpec((tm1, f_out_pad), lambda i: (i, 0)),
        compiler_params=pltpu.CompilerParams(
            dimension_semantics=("parallel",),
        ),
    )(x_p, w_p)

    out_p = pl.pallas_call(
        _prop_kernel,
        out_shape=jax.ShapeDtypeStruct((N_pad, f_out_pad), out_dtype),
        grid=(N_pad // tm2,),
        in_specs=[
            pl.BlockSpec((tm2, N_pad), lambda i: (i, 0)),
            pl.BlockSpec((N_pad, f_out_pad), lambda i: (0, 0)),
            pl.BlockSpec((1, f_out_pad), lambda i: (0, 0)),
        ],
        out_specs=pl.BlockSpec((tm2, f_out_pad), lambda i: (i, 0)),
        compiler_params=pltpu.CompilerParams(
            dimension_semantics=("parallel",),
        ),
    )(adj_p, support, b_p)

    return out_p[:N, :f_out]


# speedup vs baseline: 3.3049x; 1.0015x over previous
import jax
import jax.numpy as jnp
from jax.experimental import pallas as pl
from jax.experimental.pallas import tpu as pltpu


def _round_up(a: int, b: int) -> int:
    return ((a + b - 1) // b) * b


def _xw_kernel(x_ref, w_ref, s_ref):
    s_ref[...] = jnp.dot(
        x_ref[...].astype(jnp.bfloat16),
        w_ref[...].astype(jnp.bfloat16),
        preferred_element_type=jnp.float32,
    ).astype(s_ref.dtype)


def _prop_kernel(adj_a_ref, adj_b_ref, s_a_ref, s_b_ref, b_ref, out_ref):
    acc = jnp.dot(
        adj_a_ref[...].astype(jnp.bfloat16),
        s_a_ref[...],
        preferred_element_type=jnp.float32,
    )
    acc += jnp.dot(
        adj_b_ref[...].astype(jnp.bfloat16),
        s_b_ref[...],
        preferred_element_type=jnp.float32,
    )
    out_ref[...] = (acc + b_ref[...]).astype(out_ref.dtype)


def kernel(x, adj, weight, bias):
    N, f_in = x.shape
    f_in_w, f_out = weight.shape
    assert f_in == f_in_w, "weight shape mismatch"
    assert adj.shape == (N, N), "adj must be [N, N]"

    out_dtype = x.dtype

    N_pad = _round_up(N, 256)
    f_out_pad = _round_up(f_out, 128)

    x_p = jnp.pad(x, ((0, N_pad - N), (0, 0)))
    adj_p = jnp.pad(adj, ((0, N_pad - N), (0, N_pad - N)))
    w_p = jnp.pad(weight, ((0, 0), (0, f_out_pad - f_out)))
    b = bias if bias is not None else jnp.zeros((f_out,), out_dtype)
    b_p = jnp.pad(b, (0, f_out_pad - f_out)).reshape(1, f_out_pad).astype(jnp.float32)

    tm1 = 512 if N_pad % 512 == 0 else 256
    tm2 = 256

    support = pl.pallas_call(
        _xw_kernel,
        out_shape=jax.ShapeDtypeStruct((N_pad, f_out_pad), jnp.bfloat16),
        grid=(N_pad // tm1,),
        in_specs=[
            pl.BlockSpec((tm1, f_in), lambda i: (i, 0)),
            pl.BlockSpec((f_in, f_out_pad), lambda i: (0, 0)),
        ],
        out_specs=pl.BlockSpec((tm1, f_out_pad), lambda i: (i, 0)),
        compiler_params=pltpu.CompilerParams(
            dimension_semantics=("parallel",),
        ),
    )(x_p, w_p)

    kh = N_pad // 2
    out_p = pl.pallas_call(
        _prop_kernel,
        out_shape=jax.ShapeDtypeStruct((N_pad, f_out_pad), out_dtype),
        grid=(N_pad // tm2,),
        in_specs=[
            pl.BlockSpec((tm2, kh), lambda i: (i, 0)),
            pl.BlockSpec((tm2, kh), lambda i: (i, 1)),
            pl.BlockSpec((kh, f_out_pad), lambda i: (0, 0)),
            pl.BlockSpec((kh, f_out_pad), lambda i: (1, 0)),
            pl.BlockSpec((1, f_out_pad), lambda i: (0, 0)),
        ],
        out_specs=pl.BlockSpec((tm2, f_out_pad), lambda i: (i, 0)),
        compiler_params=pltpu.CompilerParams(
            dimension_semantics=("parallel",),
        ),
    )(adj_p, adj_p, support, support, b_p)

    return out_p[:N, :f_out]


# speedup vs baseline: 3.6613x; 1.1079x over previous
import jax
import jax.numpy as jnp
from jax.experimental import pallas as pl
from jax.experimental.pallas import tpu as pltpu


def _round_up(a: int, b: int) -> int:
    return ((a + b - 1) // b) * b


def _xw_kernel(x_ref, w_ref, s_ref):
    s_ref[...] = jnp.dot(
        x_ref[...].astype(jnp.bfloat16),
        w_ref[...].astype(jnp.bfloat16),
        preferred_element_type=jnp.float32,
    ).astype(s_ref.dtype)


def _prop_kernel(adj_a_ref, adj_b_ref, s_a_ref, s_b_ref, b_ref, out_ref):
    acc = jnp.dot(
        adj_a_ref[...].astype(jnp.bfloat16),
        s_a_ref[...],
        preferred_element_type=jnp.float32,
    )
    acc += jnp.dot(
        adj_b_ref[...].astype(jnp.bfloat16),
        s_b_ref[...],
        preferred_element_type=jnp.float32,
    )
    out_ref[...] = (acc + b_ref[...]).astype(out_ref.dtype)


def kernel(x, adj, weight, bias):
    N, f_in = x.shape
    f_in_w, f_out = weight.shape
    assert f_in == f_in_w, "weight shape mismatch"
    assert adj.shape == (N, N), "adj must be [N, N]"

    out_dtype = x.dtype

    N_pad = _round_up(N, 256)
    f_out_pad = _round_up(f_out, 128)

    x_p = jnp.pad(x, ((0, N_pad - N), (0, 0)))
    adj_p = jnp.pad(adj, ((0, N_pad - N), (0, N_pad - N)))
    w_p = jnp.pad(weight, ((0, 0), (0, f_out_pad - f_out)))
    b = bias if bias is not None else jnp.zeros((f_out,), out_dtype)
    b_p = jnp.pad(b, (0, f_out_pad - f_out)).reshape(1, f_out_pad).astype(jnp.float32)

    tm1 = 512 if N_pad % 512 == 0 else 256
    tm2 = 512

    support = pl.pallas_call(
        _xw_kernel,
        out_shape=jax.ShapeDtypeStruct((N_pad, f_out_pad), jnp.bfloat16),
        grid=(N_pad // tm1,),
        in_specs=[
            pl.BlockSpec((tm1, f_in), lambda i: (i, 0)),
            pl.BlockSpec((f_in, f_out_pad), lambda i: (0, 0)),
        ],
        out_specs=pl.BlockSpec((tm1, f_out_pad), lambda i: (i, 0)),
        compiler_params=pltpu.CompilerParams(
            dimension_semantics=("parallel",),
        ),
    )(x_p, w_p)

    kh = N_pad // 2
    out_p = pl.pallas_call(
        _prop_kernel,
        out_shape=jax.ShapeDtypeStruct((N_pad, f_out_pad), out_dtype),
        grid=(N_pad // tm2,),
        in_specs=[
            pl.BlockSpec((tm2, kh), lambda i: (i, 0)),
            pl.BlockSpec((tm2, kh), lambda i: (i, 1)),
            pl.BlockSpec((kh, f_out_pad), lambda i: (0, 0)),
            pl.BlockSpec((kh, f_out_pad), lambda i: (1, 0)),
            pl.BlockSpec((1, f_out_pad), lambda i: (0, 0)),
        ],
        out_specs=pl.BlockSpec((tm2, f_out_pad), lambda i: (i, 0)),
        compiler_params=pltpu.CompilerParams(
            dimension_semantics=("parallel",),
        ),
    )(adj_p, adj_p, support, support, b_p)

    return out_p[:N, :f_out]
